# baseline (speedup 1.0000x reference)
import functools

import jax
import jax.numpy as jnp
from jax import lax
from jax.experimental import pallas as pl
from jax.experimental.pallas import tpu as pltpu
from jax.experimental.pallas import tpu_sc as plsc

NC, NS, L = 2, 16, 16
NW = NC * NS
B, S, D = 4, 2048, 1024
PPT = S // NW
CH = 8
NCH = PPT // CH
NV = D // L
MR = B * CH

_mesh = plsc.VectorSubcoreMesh(
    core_axis_name="c", subcore_axis_name="s", num_cores=NC, num_subcores=NS
)


@functools.partial(
    pl.kernel,
    out_type=jax.ShapeDtypeStruct((B, S, D), jnp.float32),
    mesh=_mesh,
    scratch_types=[
        pltpu.VMEM((NCH, 128), jnp.int32),
        pltpu.VMEM((2, CH, D), jnp.float32),
        pltpu.VMEM((2, MR, D), jnp.float32),
        [pltpu.SemaphoreType.DMA] * 2,
        [pltpu.SemaphoreType.DMA] * 2,
        [pltpu.SemaphoreType.DMA] * 2,
    ],
)
def _emb_kernel(x_hbm, emb_hbm, pos_hbm, out_hbm, ids_v, pos_v, tok_v,
                gs, osems, psems):
    wid = lax.axis_index("s") * NC + lax.axis_index("c")
    pbase = wid * PPT
    gdesc = [None, None]
    odesc = [[None] * B, [None] * B]
    pdesc = [None, None]

    pltpu.sync_copy(x_hbm.at[wid], ids_v)

    def start_gather(c):
        q = c % 2
        gdesc[q] = pltpu.async_copy(
            emb_hbm.at[ids_v.at[c, pl.ds(0, MR)]], tok_v.at[q], gs[q]
        )

    def start_pos(c):
        pdesc[c % 2] = pltpu.async_copy(
            pos_hbm.at[pl.ds(pbase + c * CH, CH)],
            pos_v.at[c % 2],
            psems[c % 2],
        )

    def quad_add(q):
        def row_body(r, carry):
            for j in range(NV):
                sl = pl.ds(j * L, L)
                pvec = pos_v[q, r, sl]
                for b in range(B):
                    plsc.addupdate(tok_v.at[q, b * CH + r, sl], pvec)
            return carry
        lax.fori_loop(0, CH, row_body, 0)

    start_pos(0)
    start_gather(0)
    for c in range(NCH):
        q = c % 2
        if c + 1 < NCH:
            nq = (c + 1) % 2
            start_pos(c + 1)
            if odesc[nq][0] is not None:
                odesc[nq][0].wait()
            start_gather(c + 1)
        pdesc[q].wait()
        gdesc[q].wait()
        quad_add(q)
        odesc[q][0] = pltpu.async_copy(
            tok_v.at[q].reshape(B, CH, D),
            out_hbm.at[:, pl.ds(pbase + c * CH, CH)],
            osems[q],
        )
    odesc[0][0].wait()
    odesc[1][0].wait()


def kernel(x, emb, pos_emb):
    xm = jnp.asarray(x, jnp.int32).reshape(B, NW, NCH, CH)
    xm = xm.transpose(1, 2, 0, 3).reshape(NW, NCH, MR)
    xm = jnp.pad(xm, ((0, 0), (0, 0), (0, 128 - MR)))
    return _emb_kernel(xm, emb, pos_emb)
